# baseline (speedup 1.0000x reference)
import functools

import jax
import jax.numpy as jnp
from jax import lax
from jax.experimental import pallas as pl
from jax.experimental.pallas import tpu as pltpu
from jax.experimental.pallas import tpu_sc as plsc

B = 16384
L = 200
D = 64
G = 104
NDATA = 100
NC = 2
NS = 16
NW = NC * NS
GRP = 8
GL = GRP * G
FL = 2 * G
ROWS = B * L // (NDATA * GRP)
RPW = ROWS // NW
BLK = 8
STRIPE = FL // NS


def _sc_body(ids_hbm, table_hbm, out_hbm,
             ibuf, accv, rtmp, rsum, acc_sh, gsem):
    c = lax.axis_index("c")
    s = lax.axis_index("s")
    wid = s * NC + c
    base = wid * RPW

    pltpu.sync_copy(ids_hbm.at[pl.ds(base, BLK)], ibuf)
    pltpu.async_copy(table_hbm.at[ibuf.at[0]], accv, gsem).wait()

    @pl.loop(1, BLK)
    def _sup0(t):
        pltpu.async_copy(table_hbm.at[ibuf.at[t]], accv, gsem, add=True).wait()

    @pl.loop(1, RPW // BLK)
    def _blk(kb):
        pltpu.sync_copy(ids_hbm.at[pl.ds(base + kb * BLK, BLK)], ibuf)

        @pl.loop(0, BLK)
        def _sup(t):
            pltpu.async_copy(table_hbm.at[ibuf.at[t]], accv, gsem,
                             add=True).wait()

    @pl.loop(0, FL)
    def _fold(r):
        for k2 in range(D // 16):
            sl = (pl.ds(k2 * 16, 16),)
            accv[r, sl[0]] = (accv[r, sl[0]] + accv[FL + r, sl[0]]
                              + accv[2 * FL + r, sl[0]]
                              + accv[3 * FL + r, sl[0]])

    pltpu.sync_copy(accv.at[pl.ds(0, FL)], acc_sh.at[pl.ds(s * FL, FL)])
    plsc.subcore_barrier()

    pltpu.sync_copy(acc_sh.at[pl.ds(s * STRIPE, STRIPE)], rsum)

    @pl.loop(1, NS)
    def _red(p):
        pltpu.sync_copy(acc_sh.at[pl.ds(p * FL + s * STRIPE, STRIPE)], rtmp)
        for row in range(STRIPE):
            for k2 in range(D // 16):
                sl = (row, pl.ds(k2 * 16, 16))
                rsum[sl] = rsum[sl] + rtmp[sl]

    pltpu.sync_copy(rsum, out_hbm.at[c, pl.ds(s * STRIPE, STRIPE)])


_sc_embed = functools.partial(
    pl.kernel,
    out_type=jax.ShapeDtypeStruct((NC, FL, D), jnp.float32),
    mesh=plsc.VectorSubcoreMesh(
        core_axis_name="c", subcore_axis_name="s",
        num_cores=NC, num_subcores=NS),
    compiler_params=pltpu.CompilerParams(use_tc_tiling_on_sc=False),
    scratch_types=[
        pltpu.VMEM((BLK, GL), jnp.int32),
        pltpu.VMEM((GL, D), jnp.float32),
        pltpu.VMEM((STRIPE, D), jnp.float32),
        pltpu.VMEM((STRIPE, D), jnp.float32),
        pltpu.VMEM_SHARED((NS * FL, D), jnp.float32),
        pltpu.SemaphoreType.DMA,
    ],
)(_sc_body)


def _tc_combine(p_ref, o_ref):
    half = (p_ref[0] + p_ref[1]) * jnp.float32(1.0 / B)
    o_ref[...] = jnp.concatenate(
        [half[:NDATA], half[G:G + NDATA]], axis=0)


def kernel(token_ids, embedding_table):
    ids = token_ids.reshape(-1, NDATA)
    ids = jnp.pad(ids, ((0, 0), (0, G - NDATA)))
    ids = ids.reshape(ROWS, GL)

    partial = _sc_embed(ids, embedding_table)

    return pl.pallas_call(
        _tc_combine,
        out_shape=jax.ShapeDtypeStruct((L, D), jnp.float32),
    )(partial)


# speedup vs baseline: 1.0079x; 1.0079x over previous
import functools

import jax
import jax.numpy as jnp
from jax import lax
from jax.experimental import pallas as pl
from jax.experimental.pallas import tpu as pltpu
from jax.experimental.pallas import tpu_sc as plsc

B = 16384
L = 200
D = 64
G = 104
NDATA = 100
NC = 2
NS = 16
NW = NC * NS
GRP = 8
GL = GRP * G
FL = 2 * G
ROWS = B * L // (NDATA * GRP)
RPW = ROWS // NW
BLK = 8
STRIPE = FL // NS


def _sc_body(ids_hbm, table_hbm, out_hbm,
             ibuf, accA, accB, gsemA, gsemB, isem):
    c = lax.axis_index("c")
    s = lax.axis_index("s")
    wid = s * NC + c
    base = wid * RPW

    def fire(h, t, dst, sem, add):
        pltpu.async_copy(table_hbm.at[ibuf.at[h, t]], dst, sem, add=add)

    def drain(dst, sem):
        pltpu.make_async_copy(table_hbm.at[ibuf.at[0, 0]], dst, sem).wait()

    pltpu.sync_copy(ids_hbm.at[pl.ds(base, BLK)], ibuf.at[0])
    pltpu.async_copy(ids_hbm.at[pl.ds(base + BLK, BLK)], ibuf.at[1], isem)
    fire(0, 0, accA, gsemA, False)
    fire(0, 1, accB, gsemB, False)
    for j in range(1, BLK // 2):
        drain(accA, gsemA)
        fire(0, 2 * j, accA, gsemA, True)
        drain(accB, gsemB)
        fire(0, 2 * j + 1, accB, gsemB, True)

    @pl.loop(1, RPW // BLK)
    def _blk(kb):
        h = lax.rem(kb, 2)
        pltpu.make_async_copy(
            ids_hbm.at[pl.ds(base, BLK)], ibuf.at[0], isem).wait()
        drain(accA, gsemA)
        fire(h, 0, accA, gsemA, True)
        drain(accB, gsemB)
        fire(h, 1, accB, gsemB, True)

        @pl.when(kb < RPW // BLK - 1)
        def _pf():
            pltpu.async_copy(ids_hbm.at[pl.ds(base + (kb + 1) * BLK, BLK)],
                             ibuf.at[1 - h], isem)

        for j in range(1, BLK // 2):
            drain(accA, gsemA)
            fire(h, 2 * j, accA, gsemA, True)
            drain(accB, gsemB)
            fire(h, 2 * j + 1, accB, gsemB, True)

    drain(accA, gsemA)
    drain(accB, gsemB)

    @pl.loop(0, FL)
    def _fold(r):
        for k2 in range(D // 16):
            sl = pl.ds(k2 * 16, 16)
            accA[r, sl] = (accA[r, sl] + accA[FL + r, sl]
                           + accA[2 * FL + r, sl] + accA[3 * FL + r, sl]
                           + accB[r, sl] + accB[FL + r, sl]
                           + accB[2 * FL + r, sl] + accB[3 * FL + r, sl])

    pltpu.sync_copy(accA.at[pl.ds(0, FL)], out_hbm.at[wid])


_sc_embed = functools.partial(
    pl.kernel,
    out_type=jax.ShapeDtypeStruct((NW, FL, D), jnp.float32),
    mesh=plsc.VectorSubcoreMesh(
        core_axis_name="c", subcore_axis_name="s",
        num_cores=NC, num_subcores=NS),
    compiler_params=pltpu.CompilerParams(use_tc_tiling_on_sc=False),
    scratch_types=[
        pltpu.VMEM((2, BLK, GL), jnp.int32),
        pltpu.VMEM((GL, D), jnp.float32),
        pltpu.VMEM((GL, D), jnp.float32),
        pltpu.SemaphoreType.DMA,
        pltpu.SemaphoreType.DMA,
        pltpu.SemaphoreType.DMA,
    ],
)(_sc_body)


def _tc_combine(p_ref, o_ref):
    half = jnp.sum(p_ref[...], axis=0) * jnp.float32(1.0 / B)
    o_ref[...] = jnp.concatenate(
        [half[:NDATA], half[G:G + NDATA]], axis=0)


def kernel(token_ids, embedding_table):
    ids = token_ids.reshape(-1, NDATA)
    ids = jnp.pad(ids, ((0, 0), (0, G - NDATA)))
    ids = ids.reshape(ROWS, GL)

    partial = _sc_embed(ids, embedding_table)

    return pl.pallas_call(
        _tc_combine,
        out_shape=jax.ShapeDtypeStruct((L, D), jnp.float32),
    )(partial)


# speedup vs baseline: 3.4191x; 3.3924x over previous
import functools

import jax
import jax.numpy as jnp
from jax import lax
from jax.experimental import pallas as pl
from jax.experimental.pallas import tpu as pltpu
from jax.experimental.pallas import tpu_sc as plsc

B = 16384
L = 200
D = 64
NC = 2
NS = 16
NW = NC * NS
GL = 400
ROWS = B * L // GL
RPW = ROWS // NW
BLK = 8
NBLK = RPW // BLK


def _sc_body(ids_hbm, table_hbm, out_hbm, ibuf, gbufA, gbufB, acc,
             gsemA, gsemB, isem):
    c = lax.axis_index("c")
    s = lax.axis_index("s")
    wid = s * NC + c
    base = wid * RPW

    def fire(h, t, dst, sem):
        pltpu.async_copy(table_hbm.at[ibuf.at[h, t]], dst, sem)

    def drain(dst, sem):
        pltpu.make_async_copy(table_hbm.at[ibuf.at[0, 0]], dst, sem).wait()

    def accum(buf):
        @pl.loop(0, L)
        def _acc(p):
            for k2 in range(D // 16):
                sl = pl.ds(k2 * 16, 16)
                acc[p, sl] = acc[p, sl] + buf[p, sl] + buf[L + p, sl]

    pltpu.sync_copy(ids_hbm.at[pl.ds(base, BLK)], ibuf.at[0])

    @pl.loop(0, L)
    def _zero(p):
        for k2 in range(D // 16):
            acc[p, pl.ds(k2 * 16, 16)] = jnp.zeros((16,), jnp.float32)

    fire(0, 0, gbufA, gsemA)

    @pl.loop(0, NBLK)
    def _blk(kb):
        h = lax.rem(kb, 2)
        fire(h, 1, gbufB, gsemB)
        drain(gbufA, gsemA)
        accum(gbufA)
        fire(h, 2, gbufA, gsemA)

        @pl.when(kb < NBLK - 1)
        def _pf():
            pltpu.async_copy(
                ids_hbm.at[pl.ds(base + (kb + 1) * BLK, BLK)],
                ibuf.at[1 - h], isem)

        for j in range(3, BLK):
            if j % 2 == 1:
                drain(gbufB, gsemB)
                accum(gbufB)
                fire(h, j, gbufB, gsemB)
            else:
                drain(gbufA, gsemA)
                accum(gbufA)
                fire(h, j, gbufA, gsemA)

        drain(gbufA, gsemA)
        accum(gbufA)

        @pl.when(kb < NBLK - 1)
        def _nx():
            pltpu.make_async_copy(
                ids_hbm.at[pl.ds(base, BLK)], ibuf.at[0], isem).wait()
            fire(1 - h, 0, gbufA, gsemA)

        drain(gbufB, gsemB)
        accum(gbufB)

    pltpu.sync_copy(acc, out_hbm.at[wid])


_sc_embed = functools.partial(
    pl.kernel,
    out_type=jax.ShapeDtypeStruct((NW, L, D), jnp.float32),
    mesh=plsc.VectorSubcoreMesh(
        core_axis_name="c", subcore_axis_name="s",
        num_cores=NC, num_subcores=NS),
    compiler_params=pltpu.CompilerParams(use_tc_tiling_on_sc=False),
    scratch_types=[
        pltpu.VMEM((2, BLK, GL), jnp.int32),
        pltpu.VMEM((GL, D), jnp.float32),
        pltpu.VMEM((GL, D), jnp.float32),
        pltpu.VMEM((L, D), jnp.float32),
        pltpu.SemaphoreType.DMA,
        pltpu.SemaphoreType.DMA,
        pltpu.SemaphoreType.DMA,
    ],
)(_sc_body)


def _tc_combine(p_ref, o_ref):
    o_ref[...] = jnp.sum(p_ref[...], axis=0) * jnp.float32(1.0 / B)


def kernel(token_ids, embedding_table):
    ids = token_ids.reshape(ROWS, GL)
    partial = _sc_embed(ids, embedding_table)
    return pl.pallas_call(
        _tc_combine,
        out_shape=jax.ShapeDtypeStruct((L, D), jnp.float32),
    )(partial)


# speedup vs baseline: 3.4200x; 1.0003x over previous
import functools

import jax
import jax.numpy as jnp
from jax import lax
from jax.experimental import pallas as pl
from jax.experimental.pallas import tpu as pltpu
from jax.experimental.pallas import tpu_sc as plsc

B = 16384
L = 200
D = 64
NC = 2
NS = 16
NW = NC * NS
GL = 400
ROWS = B * L // GL
RPW = ROWS // NW
BLK = 8
NBLK = RPW // BLK


def _sc_body(ids_hbm, table_hbm, out_hbm, ibuf, gbufA, gbufB, acc,
             gsemA, gsemB, isem):
    c = lax.axis_index("c")
    s = lax.axis_index("s")
    wid = s * NC + c
    base = wid * RPW

    def fire(h, t, dst, sem):
        pltpu.async_copy(
            table_hbm.at[ibuf.at[h, pl.ds(t * GL, GL)]], dst, sem)

    def drain(dst, sem):
        pltpu.make_async_copy(
            table_hbm.at[ibuf.at[0, pl.ds(0, GL)]], dst, sem).wait()

    def accum(buf):
        @pl.loop(0, L)
        def _acc(p):
            for k2 in range(D // 16):
                sl = pl.ds(k2 * 16, 16)
                acc[p, sl] = acc[p, sl] + buf[p, sl] + buf[L + p, sl]

    pltpu.sync_copy(ids_hbm.at[pl.ds(base * GL, BLK * GL)], ibuf.at[0])

    @pl.loop(0, L)
    def _zero(p):
        for k2 in range(D // 16):
            acc[p, pl.ds(k2 * 16, 16)] = jnp.zeros((16,), jnp.float32)

    fire(0, 0, gbufA, gsemA)

    @pl.loop(0, NBLK)
    def _blk(kb):
        h = lax.rem(kb, 2)
        fire(h, 1, gbufB, gsemB)
        drain(gbufA, gsemA)
        accum(gbufA)
        fire(h, 2, gbufA, gsemA)

        @pl.when(kb < NBLK - 1)
        def _pf():
            pltpu.async_copy(
                ids_hbm.at[pl.ds((base + (kb + 1) * BLK) * GL, BLK * GL)],
                ibuf.at[1 - h], isem)

        for j in range(3, BLK):
            if j % 2 == 1:
                drain(gbufB, gsemB)
                accum(gbufB)
                fire(h, j, gbufB, gsemB)
            else:
                drain(gbufA, gsemA)
                accum(gbufA)
                fire(h, j, gbufA, gsemA)

        drain(gbufA, gsemA)
        accum(gbufA)

        @pl.when(kb < NBLK - 1)
        def _nx():
            pltpu.make_async_copy(
                ids_hbm.at[pl.ds(0, BLK * GL)], ibuf.at[0], isem).wait()
            fire(1 - h, 0, gbufA, gsemA)

        drain(gbufB, gsemB)
        accum(gbufB)

    pltpu.sync_copy(acc, out_hbm.at[wid])


_sc_embed = functools.partial(
    pl.kernel,
    out_type=jax.ShapeDtypeStruct((NW, L, D), jnp.float32),
    mesh=plsc.VectorSubcoreMesh(
        core_axis_name="c", subcore_axis_name="s",
        num_cores=NC, num_subcores=NS),
    compiler_params=pltpu.CompilerParams(use_tc_tiling_on_sc=False),
    scratch_types=[
        pltpu.VMEM((2, BLK * GL), jnp.int32),
        pltpu.VMEM((GL, D), jnp.float32),
        pltpu.VMEM((GL, D), jnp.float32),
        pltpu.VMEM((L, D), jnp.float32),
        pltpu.SemaphoreType.DMA,
        pltpu.SemaphoreType.DMA,
        pltpu.SemaphoreType.DMA,
    ],
)(_sc_body)


def _tc_combine(p_ref, o_ref):
    o_ref[...] = jnp.sum(p_ref[...], axis=0) * jnp.float32(1.0 / B)


def kernel(token_ids, embedding_table):
    ids = token_ids.reshape(-1)
    partial = _sc_embed(ids, embedding_table)
    return pl.pallas_call(
        _tc_combine,
        out_shape=jax.ShapeDtypeStruct((L, D), jnp.float32),
    )(partial)


# speedup vs baseline: 3.4351x; 1.0044x over previous
import functools

import jax
import jax.numpy as jnp
from jax import lax
from jax.experimental import pallas as pl
from jax.experimental.pallas import tpu as pltpu
from jax.experimental.pallas import tpu_sc as plsc

B = 16384
L = 200
D = 64
NC = 2
NS = 16
NW = NC * NS
RPW = B // NW
BLK = 16
NBLK = RPW // BLK


def _sc_body(ids_hbm, table_hbm, out_hbm, ibuf, gbufA, gbufB, gbufC, gbufD,
             acc, semA, semB, semC, semD, isem):
    c = lax.axis_index("c")
    s = lax.axis_index("s")
    wid = s * NC + c
    base = wid * RPW

    def fire(h, t, dst, sem):
        pltpu.async_copy(table_hbm.at[ibuf.at[h, t]], dst, sem)

    def drain(dst, sem):
        pltpu.make_async_copy(table_hbm.at[ibuf.at[0, 0]], dst, sem).wait()

    def accum2(bx, by):
        @pl.loop(0, L)
        def _acc(p):
            for k2 in range(D // 16):
                sl = pl.ds(k2 * 16, 16)
                acc[p, sl] = acc[p, sl] + bx[p, sl] + by[p, sl]

    pltpu.sync_copy(ids_hbm.at[pl.ds(base, BLK)], ibuf.at[0])

    @pl.loop(0, L)
    def _zero(p):
        for k2 in range(D // 16):
            acc[p, pl.ds(k2 * 16, 16)] = jnp.zeros((16,), jnp.float32)

    fire(0, 0, gbufA, semA)
    fire(0, 1, gbufB, semB)
    fire(0, 2, gbufC, semC)
    fire(0, 3, gbufD, semD)

    @pl.loop(0, NBLK)
    def _blk(kb):
        h = lax.rem(kb, 2)

        drain(gbufA, semA)
        drain(gbufB, semB)
        accum2(gbufA, gbufB)
        fire(h, 4, gbufA, semA)
        fire(h, 5, gbufB, semB)

        @pl.when(kb < NBLK - 1)
        def _pf():
            pltpu.async_copy(ids_hbm.at[pl.ds(base + (kb + 1) * BLK, BLK)],
                             ibuf.at[1 - h], isem)

        for j in range(1, 6):
            if j % 2 == 1:
                drain(gbufC, semC)
                drain(gbufD, semD)
                accum2(gbufC, gbufD)
                fire(h, 2 * j + 4, gbufC, semC)
                fire(h, 2 * j + 5, gbufD, semD)
            else:
                drain(gbufA, semA)
                drain(gbufB, semB)
                accum2(gbufA, gbufB)
                fire(h, 2 * j + 4, gbufA, semA)
                fire(h, 2 * j + 5, gbufB, semB)

        drain(gbufA, semA)
        drain(gbufB, semB)
        accum2(gbufA, gbufB)

        @pl.when(kb < NBLK - 1)
        def _nx0():
            pltpu.make_async_copy(
                ids_hbm.at[pl.ds(base, BLK)], ibuf.at[0], isem).wait()
            fire(1 - h, 0, gbufA, semA)
            fire(1 - h, 1, gbufB, semB)

        drain(gbufC, semC)
        drain(gbufD, semD)
        accum2(gbufC, gbufD)

        @pl.when(kb < NBLK - 1)
        def _nx1():
            fire(1 - h, 2, gbufC, semC)
            fire(1 - h, 3, gbufD, semD)

    pltpu.sync_copy(acc, out_hbm.at[wid])


_sc_embed = functools.partial(
    pl.kernel,
    out_type=jax.ShapeDtypeStruct((NW, L, D), jnp.float32),
    mesh=plsc.VectorSubcoreMesh(
        core_axis_name="c", subcore_axis_name="s",
        num_cores=NC, num_subcores=NS),
    compiler_params=pltpu.CompilerParams(use_tc_tiling_on_sc=False),
    scratch_types=[
        pltpu.VMEM((2, BLK, L), jnp.int32),
        pltpu.VMEM((L, D), jnp.float32),
        pltpu.VMEM((L, D), jnp.float32),
        pltpu.VMEM((L, D), jnp.float32),
        pltpu.VMEM((L, D), jnp.float32),
        pltpu.VMEM((L, D), jnp.float32),
        pltpu.SemaphoreType.DMA,
        pltpu.SemaphoreType.DMA,
        pltpu.SemaphoreType.DMA,
        pltpu.SemaphoreType.DMA,
        pltpu.SemaphoreType.DMA,
    ],
)(_sc_body)


def _tc_combine(p_ref, o_ref):
    o_ref[...] = jnp.sum(p_ref[...], axis=0) * jnp.float32(1.0 / B)


def kernel(token_ids, embedding_table):
    partial = _sc_embed(token_ids, embedding_table)
    return pl.pallas_call(
        _tc_combine,
        out_shape=jax.ShapeDtypeStruct((L, D), jnp.float32),
    )(partial)
